# baseline (speedup 1.0000x reference)
import functools

import jax
import jax.numpy as jnp
from jax.experimental import pallas as pl
from jax.experimental.pallas import tpu as pltpu

ES = 32
BIN = 50
FD = ES + ES // 8 + 1
PD = 5 * ES + ES // 8 + 1
FEAT_DIM = 1 + 1 + 9 + 3 + BIN * 3 + 1001

TILE = 4096


def _leaky(x):
    return jnp.where(x >= 0, x, 0.01 * x)


def _dn(a, b):
    return jax.lax.dot_general(a, b, (((1,), (1,)), ((), ())),
                               preferred_element_type=jnp.float32)


def _iota2(shape, dim):
    return jax.lax.broadcasted_iota(jnp.int32, shape, dim)


def _eye(n):
    return (_iota2((n, n), 0) == _iota2((n, n), 1)).astype(jnp.float32)


def _dot(a, b):
    return jnp.dot(a, b, preferred_element_type=jnp.float32)


def _col(row, n):
    w = jnp.where(_iota2((n, n), 0) == _iota2((n, n), 1),
                  jnp.broadcast_to(row, (n, n)), 0.0)
    return jnp.sum(w, axis=1, keepdims=True)


def _body(f_ref, te_ref, tab_ref, ce_ref, oe_ref, je_ref, wf_ref, bf_ref,
          wf2_ref, bf2_ref, ws_ref, bs_ref, wh_ref, bh_ref, wp_ref, bp_ref,
          o_ref,
          a102_s, v3a_s, w2a_s, m1_s, g70_s, wp2_s, wp4_s, wp5a_s):
    f32 = jnp.float32

    @pl.when(pl.program_id(0) == 0)
    def _prep():
        wf = wf_ref[...]
        wp = wp_ref[...]

        a_colt = _dn(wf[:, 0:ES], ce_ref[...])
        a_opt = _dn(wf[:, ES:ES + 4], oe_ref[...])
        a102_s[...] = jnp.zeros((3 * FD, 102), f32)
        a102_s[0:FD, 0:30] = a_colt
        a102_s[FD:2 * FD, 30:60] = a_colt
        a102_s[2 * FD:3 * FD, 60:90] = a_colt
        a102_s[0:FD, 90:94] = a_opt
        a102_s[FD:2 * FD, 94:98] = a_opt
        a102_s[2 * FD:3 * FD, 98:102] = a_opt
        a_val = wf[:, FD - 1:FD]
        bf_col = _col(bf_ref[...][None, :], FD)
        v3a_s[...] = jnp.zeros((3 * FD, 4), f32)
        v3a_s[0:FD, 0:1] = a_val
        v3a_s[FD:2 * FD, 1:2] = a_val
        v3a_s[2 * FD:3 * FD, 2:3] = a_val
        v3a_s[0:FD, 3:4] = bf_col
        v3a_s[FD:2 * FD, 3:4] = bf_col
        v3a_s[2 * FD:3 * FD, 3:4] = bf_col

        wf2 = wf2_ref[...]
        bf2_col = _col(bf2_ref[...][None, :], FD)
        w2a_s[...] = jnp.zeros((3 * FD, 3 * FD + 1), f32)
        w2a_s[0:FD, 0:FD] = wf2
        w2a_s[FD:2 * FD, FD:2 * FD] = wf2
        w2a_s[2 * FD:3 * FD, 2 * FD:3 * FD] = wf2
        w2a_s[0:FD, 3 * FD:3 * FD + 1] = bf2_col
        w2a_s[FD:2 * FD, 3 * FD:3 * FD + 1] = bf2_col
        w2a_s[2 * FD:3 * FD, 3 * FD:3 * FD + 1] = bf2_col

        m1_s[...] = jnp.zeros((128, FEAT_DIM + 1), f32)
        rr = _iota2((3 * BIN, BIN), 0)
        cc = _iota2((3 * BIN, BIN), 1)
        wh = wh_ref[...]
        bh_col = _col(bh_ref[...][None, :], ES)
        for j in range(3):
            ej = (rr == 3 * cc + j).astype(f32)
            m1_s[ES * j:ES * (j + 1), 14:164] = _dn(wh, ej)
            m1_s[ES * j:ES * (j + 1), FEAT_DIM:FEAT_DIM + 1] = bh_col
        m1_s[96:128, 165:FEAT_DIM] = ws_ref[...]

        g70_s[:, 0:20] = _dn(wp[:, 0:ES], te_ref[...])
        g70_s[:, 20:60] = _dn(wp[:, ES + FD:2 * ES + FD], je_ref[...])
        g70_s[:, 60:70] = _dn(wp[:, 2 * ES + FD:3 * ES + FD], tab_ref[...])
        wp2_s[...] = wp[:, ES:ES + FD]
        wp4 = wp[:, 2 * ES + FD:3 * ES + FD]
        wp4_s[...] = wp4
        bp2_row = bp_ref[...][None, :] + _dn(bs_ref[...][None, :], wp4)
        wp5a_s[:, 0:ES] = wp[:, PD - ES:PD]
        wp5a_s[:, ES:ES + 1] = _col(bp2_row, PD)

    f = f_ref[...]
    t = f.shape[1]

    type_id = f[0:1, :].astype(jnp.int32)
    join_id = f[1:2, :].astype(jnp.int32)
    cols = f[2:5, :].astype(jnp.int32)
    ops = f[5:8, :].astype(jnp.int32)
    vals = f[8:11, :]
    m = f[11:14, :]
    tab_id = f[164:165, :].astype(jnp.int32)
    ones1 = jnp.ones((1, t), f32)

    i102 = _iota2((102, t), 0)
    oh102 = (
        (i102 == cols[0:1, :]).astype(f32)
        + (i102 == cols[1:2, :] + 30).astype(f32)
        + (i102 == cols[2:3, :] + 60).astype(f32)
        + (i102 == ops[0:1, :] + 90).astype(f32)
        + (i102 == ops[1:2, :] + 94).astype(f32)
        + (i102 == ops[2:3, :] + 98).astype(f32)
    )
    va = jnp.concatenate([vals, ones1], axis=0)
    h1 = _leaky(_dot(a102_s[...], oh102) + _dot(v3a_s[...], va))
    h1a = jnp.concatenate([h1, ones1], axis=0)
    h2 = _leaky(_dot(w2a_s[...], h1a))

    nf = m[0:1, :] + m[1:2, :] + m[2:3, :]
    zero = jnp.zeros_like(h2[0:FD, :])
    fsum = (jnp.where(m[0:1, :] != 0, h2[0:FD, :], zero)
            + jnp.where(m[1:2, :] != 0, h2[FD:2 * FD, :], zero)
            + jnp.where(m[2:3, :] != 0, h2[2 * FD:3 * FD, :], zero))
    filter_emb = fsum / nf

    fa = jnp.concatenate([f, ones1], axis=0)
    hs = _dot(m1_s[...], fa)
    zero32 = jnp.zeros_like(hs[0:ES, :])
    hist_sum = (jnp.where(m[0:1, :] != 0, hs[0:ES, :], zero32)
                + jnp.where(m[1:2, :] != 0, hs[ES:2 * ES, :], zero32)
                + jnp.where(m[2:3, :] != 0, hs[2 * ES:3 * ES, :], zero32))
    hist_emb = hist_sum / nf
    samp = hs[96:128, :]

    i70 = _iota2((70, t), 0)
    oh70 = ((i70 == type_id).astype(f32)
            + (i70 == join_id + 20).astype(f32)
            + (i70 == tab_id + 60).astype(f32))
    ha = jnp.concatenate([hist_emb, ones1], axis=0)
    pre = (_dot(g70_s[...], oh70)
           + _dot(wp2_s[...], filter_emb)
           + _dot(wp4_s[...], samp)
           + _dot(wp5a_s[...], ha))
    o_ref[...] = _leaky(pre)


@functools.partial(jax.jit, static_argnames=())
def kernel(feature, typeEmbed, tableEmbed, columnEmbed, opEmbed, joinEmbed,
           Wf, bf, Wf2, bf2, Ws, bs, Wh, bh, Wp, bp):
    b = feature.shape[0]
    f32 = jnp.float32

    ft = feature.T
    full = lambda s: pl.BlockSpec(s, lambda i: (0,) * len(s))
    scratch = lambda *s: pltpu.VMEM(s, f32)
    out_t = pl.pallas_call(
        _body,
        grid=(b // TILE,),
        in_specs=[
            pl.BlockSpec((FEAT_DIM, TILE), lambda i: (0, i)),
            full((20, ES)), full((10, ES)), full((30, ES)), full((4, 4)),
            full((40, ES)), full((FD, FD)), full((FD,)), full((FD, FD)),
            full((FD,)), full((ES, 1000)), full((ES,)), full((ES, BIN)),
            full((ES,)), full((PD, PD)), full((PD,)),
        ],
        out_specs=pl.BlockSpec((PD, TILE), lambda i: (0, i)),
        out_shape=jax.ShapeDtypeStruct((PD, b), f32),
        scratch_shapes=[
            scratch(3 * FD, 102), scratch(3 * FD, 4),
            scratch(3 * FD, 3 * FD + 1), scratch(128, FEAT_DIM + 1),
            scratch(PD, 70), scratch(PD, FD), scratch(PD, ES),
            scratch(PD, ES + 1),
        ],
        compiler_params=pltpu.CompilerParams(
            dimension_semantics=("arbitrary",),
        ),
    )(ft, typeEmbed, tableEmbed, columnEmbed, opEmbed, joinEmbed,
      Wf, bf, Wf2, bf2, Ws, bs, Wh, bh, Wp, bp)
    return out_t.T


# speedup vs baseline: 1.0524x; 1.0524x over previous
import functools

import jax
import jax.numpy as jnp
from jax.experimental import pallas as pl
from jax.experimental.pallas import tpu as pltpu

ES = 32
BIN = 50
FD = ES + ES // 8 + 1
PD = 5 * ES + ES // 8 + 1
FEAT_DIM = 1 + 1 + 9 + 3 + BIN * 3 + 1001

TILE = 2048


def _leaky(x):
    return jnp.where(x >= 0, x, 0.01 * x)


def _dn(a, b):
    return jax.lax.dot_general(a, b, (((1,), (1,)), ((), ())),
                               preferred_element_type=jnp.float32)


def _iota2(shape, dim):
    return jax.lax.broadcasted_iota(jnp.int32, shape, dim)


def _eye(n):
    return (_iota2((n, n), 0) == _iota2((n, n), 1)).astype(jnp.float32)


def _dot(a, b):
    return jnp.dot(a, b, preferred_element_type=jnp.float32)


def _col(row, n):
    w = jnp.where(_iota2((n, n), 0) == _iota2((n, n), 1),
                  jnp.broadcast_to(row, (n, n)), 0.0)
    return jnp.sum(w, axis=1, keepdims=True)


def _body(f_ref, te_ref, tab_ref, ce_ref, oe_ref, je_ref, wf_ref, bf_ref,
          wf2_ref, bf2_ref, ws_ref, bs_ref, wh_ref, bh_ref, wp_ref, bp_ref,
          o_ref,
          a1_s, w2a_s, m1_s, wfin_s):
    f32 = jnp.float32
    bf16 = jnp.bfloat16

    @pl.when(pl.program_id(0) == 0)
    def _prep():
        wf = wf_ref[...]
        wp = wp_ref[...]

        a_colt = _dn(wf[:, 0:ES], ce_ref[...])
        a_opt = _dn(wf[:, ES:ES + 4], oe_ref[...])
        a1_s[...] = jnp.zeros((3 * FD, 106), bf16)
        a1_s[0:FD, 0:30] = a_colt.astype(bf16)
        a1_s[FD:2 * FD, 30:60] = a_colt.astype(bf16)
        a1_s[2 * FD:3 * FD, 60:90] = a_colt.astype(bf16)
        a1_s[0:FD, 90:94] = a_opt.astype(bf16)
        a1_s[FD:2 * FD, 94:98] = a_opt.astype(bf16)
        a1_s[2 * FD:3 * FD, 98:102] = a_opt.astype(bf16)
        a_val = wf[:, FD - 1:FD].astype(bf16)
        bf_col = _col(bf_ref[...][None, :], FD).astype(bf16)
        a1_s[0:FD, 102:103] = a_val
        a1_s[FD:2 * FD, 103:104] = a_val
        a1_s[2 * FD:3 * FD, 104:105] = a_val
        a1_s[0:FD, 105:106] = bf_col
        a1_s[FD:2 * FD, 105:106] = bf_col
        a1_s[2 * FD:3 * FD, 105:106] = bf_col

        wf2 = wf2_ref[...]
        bf2_col = _col(bf2_ref[...][None, :], FD).astype(bf16)
        w2a_s[...] = jnp.zeros((3 * FD, 3 * FD + 1), bf16)
        w2a_s[0:FD, 0:FD] = wf2.astype(bf16)
        w2a_s[FD:2 * FD, FD:2 * FD] = wf2.astype(bf16)
        w2a_s[2 * FD:3 * FD, 2 * FD:3 * FD] = wf2.astype(bf16)
        w2a_s[0:FD, 3 * FD:3 * FD + 1] = bf2_col
        w2a_s[FD:2 * FD, 3 * FD:3 * FD + 1] = bf2_col
        w2a_s[2 * FD:3 * FD, 3 * FD:3 * FD + 1] = bf2_col

        m1_s[...] = jnp.zeros((128, FEAT_DIM + 1), bf16)
        rr = _iota2((3 * BIN, BIN), 0)
        cc = _iota2((3 * BIN, BIN), 1)
        wh = wh_ref[...]
        bh_col = _col(bh_ref[...][None, :], ES).astype(bf16)
        for j in range(3):
            ej = (rr == 3 * cc + j).astype(f32)
            m1_s[ES * j:ES * (j + 1), 14:164] = _dn(wh, ej).astype(bf16)
            m1_s[ES * j:ES * (j + 1), FEAT_DIM:FEAT_DIM + 1] = bh_col
        m1_s[96:128, 165:FEAT_DIM] = ws_ref[...].astype(bf16)

        wfin_s[:, 0:20] = _dn(wp[:, 0:ES], te_ref[...]).astype(bf16)
        wfin_s[:, 20:60] = _dn(wp[:, ES + FD:2 * ES + FD],
                               je_ref[...]).astype(bf16)
        wfin_s[:, 60:70] = _dn(wp[:, 2 * ES + FD:3 * ES + FD],
                               tab_ref[...]).astype(bf16)
        wfin_s[:, 70:107] = wp[:, ES:ES + FD].astype(bf16)
        wp4 = wp[:, 2 * ES + FD:3 * ES + FD]
        wfin_s[:, 107:139] = wp4.astype(bf16)
        wfin_s[:, 139:171] = wp[:, PD - ES:PD].astype(bf16)
        bp2_row = bp_ref[...][None, :] + _dn(bs_ref[...][None, :], wp4)
        wfin_s[:, 171:172] = _col(bp2_row, PD).astype(bf16)

    f = f_ref[...]
    t = f.shape[1]

    type_id = f[0:1, :].astype(jnp.int32)
    join_id = f[1:2, :].astype(jnp.int32)
    cols = f[2:5, :].astype(jnp.int32)
    ops = f[5:8, :].astype(jnp.int32)
    vals = f[8:11, :]
    m = f[11:14, :]
    tab_id = f[164:165, :].astype(jnp.int32)
    ones1b = jnp.ones((1, t), bf16)

    i102 = _iota2((102, t), 0)
    oh102 = (
        ((i102 == cols[0:1, :]) | (i102 == cols[1:2, :] + 30)
         | (i102 == cols[2:3, :] + 60) | (i102 == ops[0:1, :] + 90)
         | (i102 == ops[1:2, :] + 94) | (i102 == ops[2:3, :] + 98))
    ).astype(bf16)
    acts1 = jnp.concatenate([oh102, vals.astype(bf16), ones1b],
                            axis=0)
    h1 = _leaky(_dot(a1_s[...], acts1))
    h1a = jnp.concatenate([h1.astype(bf16), ones1b], axis=0)
    h2 = _leaky(_dot(w2a_s[...], h1a))

    nf = m[0:1, :] + m[1:2, :] + m[2:3, :]
    zero = jnp.zeros_like(h2[0:FD, :])
    fsum = (jnp.where(m[0:1, :] != 0, h2[0:FD, :], zero)
            + jnp.where(m[1:2, :] != 0, h2[FD:2 * FD, :], zero)
            + jnp.where(m[2:3, :] != 0, h2[2 * FD:3 * FD, :], zero))
    filter_emb = fsum / nf

    fa = jnp.concatenate([f.astype(bf16), ones1b], axis=0)
    hs = _dot(m1_s[...], fa)
    zero32 = jnp.zeros_like(hs[0:ES, :])
    hist_sum = (jnp.where(m[0:1, :] != 0, hs[0:ES, :], zero32)
                + jnp.where(m[1:2, :] != 0, hs[ES:2 * ES, :], zero32)
                + jnp.where(m[2:3, :] != 0, hs[2 * ES:3 * ES, :], zero32))
    hist_emb = hist_sum / nf
    samp = hs[96:128, :]

    i70 = _iota2((70, t), 0)
    oh70 = ((i70 == type_id) | (i70 == join_id + 20)
            | (i70 == tab_id + 60)).astype(bf16)
    acts3 = jnp.concatenate([oh70, filter_emb.astype(bf16),
                             samp.astype(bf16), hist_emb.astype(bf16),
                             ones1b], axis=0)
    o_ref[...] = _leaky(_dot(wfin_s[...], acts3))


@functools.partial(jax.jit, static_argnames=())
def kernel(feature, typeEmbed, tableEmbed, columnEmbed, opEmbed, joinEmbed,
           Wf, bf, Wf2, bf2, Ws, bs, Wh, bh, Wp, bp):
    b = feature.shape[0]
    f32 = jnp.float32

    ft = feature.T
    full = lambda s: pl.BlockSpec(s, lambda i: (0,) * len(s))
    scratch = lambda *s: pltpu.VMEM(s, jnp.bfloat16)
    out_t = pl.pallas_call(
        _body,
        grid=(b // TILE,),
        in_specs=[
            pl.BlockSpec((FEAT_DIM, TILE), lambda i: (0, i)),
            full((20, ES)), full((10, ES)), full((30, ES)), full((4, 4)),
            full((40, ES)), full((FD, FD)), full((FD,)), full((FD, FD)),
            full((FD,)), full((ES, 1000)), full((ES,)), full((ES, BIN)),
            full((ES,)), full((PD, PD)), full((PD,)),
        ],
        out_specs=pl.BlockSpec((PD, TILE), lambda i: (0, i)),
        out_shape=jax.ShapeDtypeStruct((PD, b), f32),
        scratch_shapes=[
            scratch(3 * FD, 106), scratch(3 * FD, 3 * FD + 1),
            scratch(128, FEAT_DIM + 1), scratch(PD, 172),
        ],
        compiler_params=pltpu.CompilerParams(
            dimension_semantics=("arbitrary",),
        ),
    )(ft, typeEmbed, tableEmbed, columnEmbed, opEmbed, joinEmbed,
      Wf, bf, Wf2, bf2, Ws, bs, Wh, bh, Wp, bp)
    return out_t.T


# speedup vs baseline: 1.1009x; 1.0460x over previous
import functools

import jax
import jax.numpy as jnp
from jax.experimental import pallas as pl
from jax.experimental.pallas import tpu as pltpu

ES = 32
BIN = 50
FD = ES + ES // 8 + 1
PD = 5 * ES + ES // 8 + 1
FEAT_DIM = 1 + 1 + 9 + 3 + BIN * 3 + 1001

TILE = 2048


def _leaky(x):
    return jnp.where(x >= 0, x, 0.01 * x)


def _dn(a, b):
    return jax.lax.dot_general(a, b, (((1,), (1,)), ((), ())),
                               preferred_element_type=jnp.float32)


def _iota2(shape, dim):
    return jax.lax.broadcasted_iota(jnp.int32, shape, dim)


def _eye(n):
    return (_iota2((n, n), 0) == _iota2((n, n), 1)).astype(jnp.float32)


def _dot(a, b):
    return jnp.dot(a, b, preferred_element_type=jnp.float32)


def _col(row, n):
    w = jnp.where(_iota2((n, n), 0) == _iota2((n, n), 1),
                  jnp.broadcast_to(row, (n, n)), 0.0)
    return jnp.sum(w, axis=1, keepdims=True)


def _body(f_ref, te_ref, tab_ref, ce_ref, oe_ref, je_ref, wf_ref, bf_ref,
          wf2_ref, bf2_ref, ws_ref, bs_ref, wh_ref, bh_ref, wp_ref, bp_ref,
          o_ref,
          a1_s, w2a_s, m1_s, wfin_s):
    f32 = jnp.float32
    bf16 = jnp.bfloat16

    @pl.when(pl.program_id(0) == 0)
    def _prep():
        wf = wf_ref[...]
        wp = wp_ref[...]

        a_colt = _dn(wf[:, 0:ES], ce_ref[...])
        a_opt = _dn(wf[:, ES:ES + 4], oe_ref[...])
        a1_s[...] = jnp.zeros((3 * FD, 106), bf16)
        a1_s[0:FD, 0:30] = a_colt.astype(bf16)
        a1_s[FD:2 * FD, 30:60] = a_colt.astype(bf16)
        a1_s[2 * FD:3 * FD, 60:90] = a_colt.astype(bf16)
        a1_s[0:FD, 90:94] = a_opt.astype(bf16)
        a1_s[FD:2 * FD, 94:98] = a_opt.astype(bf16)
        a1_s[2 * FD:3 * FD, 98:102] = a_opt.astype(bf16)
        a_val = wf[:, FD - 1:FD].astype(bf16)
        bf_col = _col(bf_ref[...][None, :], FD).astype(bf16)
        a1_s[0:FD, 102:103] = a_val
        a1_s[FD:2 * FD, 103:104] = a_val
        a1_s[2 * FD:3 * FD, 104:105] = a_val
        a1_s[0:FD, 105:106] = bf_col
        a1_s[FD:2 * FD, 105:106] = bf_col
        a1_s[2 * FD:3 * FD, 105:106] = bf_col

        wf2 = wf2_ref[...]
        bf2_col = _col(bf2_ref[...][None, :], FD).astype(bf16)
        w2a_s[...] = jnp.zeros((3 * FD, 3 * FD + 1), bf16)
        w2a_s[0:FD, 0:FD] = wf2.astype(bf16)
        w2a_s[FD:2 * FD, FD:2 * FD] = wf2.astype(bf16)
        w2a_s[2 * FD:3 * FD, 2 * FD:3 * FD] = wf2.astype(bf16)
        w2a_s[0:FD, 3 * FD:3 * FD + 1] = bf2_col
        w2a_s[FD:2 * FD, 3 * FD:3 * FD + 1] = bf2_col
        w2a_s[2 * FD:3 * FD, 3 * FD:3 * FD + 1] = bf2_col

        m1_s[...] = jnp.zeros((128, FEAT_DIM + 1), bf16)
        rr = _iota2((3 * BIN, BIN), 0)
        cc = _iota2((3 * BIN, BIN), 1)
        wh = wh_ref[...]
        bh_col = _col(bh_ref[...][None, :], ES).astype(bf16)
        for j in range(3):
            ej = (rr == 3 * cc + j).astype(f32)
            m1_s[ES * j:ES * (j + 1), 14:164] = _dn(wh, ej).astype(bf16)
            m1_s[ES * j:ES * (j + 1), FEAT_DIM:FEAT_DIM + 1] = bh_col
        m1_s[96:128, 165:FEAT_DIM] = ws_ref[...].astype(bf16)

        wfin_s[:, 0:20] = _dn(wp[:, 0:ES], te_ref[...]).astype(bf16)
        wfin_s[:, 20:60] = _dn(wp[:, ES + FD:2 * ES + FD],
                               je_ref[...]).astype(bf16)
        wfin_s[:, 60:70] = _dn(wp[:, 2 * ES + FD:3 * ES + FD],
                               tab_ref[...]).astype(bf16)
        wfin_s[:, 70:107] = wp[:, ES:ES + FD].astype(bf16)
        wp4 = wp[:, 2 * ES + FD:3 * ES + FD]
        wfin_s[:, 107:139] = wp4.astype(bf16)
        wfin_s[:, 139:171] = wp[:, PD - ES:PD].astype(bf16)
        bp2_row = bp_ref[...][None, :] + _dn(bs_ref[...][None, :], wp4)
        wfin_s[:, 171:172] = _col(bp2_row, PD).astype(bf16)

    f = f_ref[...]
    t = f.shape[1]

    type_id = f[0:1, :].astype(jnp.int32)
    join_id = f[1:2, :].astype(jnp.int32)
    cols = f[2:5, :].astype(jnp.int32)
    ops = f[5:8, :].astype(jnp.int32)
    vals = f[8:11, :]
    m = f[11:14, :]
    tab_id = f[164:165, :].astype(jnp.int32)
    ones1b = jnp.ones((1, t), bf16)

    i30 = _iota2((30, t), 0)
    i4 = _iota2((4, t), 0)
    acts1 = jnp.concatenate(
        [(i30 == cols[0:1, :]).astype(bf16),
         (i30 == cols[1:2, :]).astype(bf16),
         (i30 == cols[2:3, :]).astype(bf16),
         (i4 == ops[0:1, :]).astype(bf16),
         (i4 == ops[1:2, :]).astype(bf16),
         (i4 == ops[2:3, :]).astype(bf16),
         vals.astype(bf16), ones1b], axis=0)
    h1 = _leaky(_dot(a1_s[...], acts1))
    h1a = jnp.concatenate([h1.astype(bf16), ones1b], axis=0)
    h2 = _leaky(_dot(w2a_s[...], h1a))

    nf = m[0:1, :] + m[1:2, :] + m[2:3, :]
    zero = jnp.zeros_like(h2[0:FD, :])
    fsum = (jnp.where(m[0:1, :] != 0, h2[0:FD, :], zero)
            + jnp.where(m[1:2, :] != 0, h2[FD:2 * FD, :], zero)
            + jnp.where(m[2:3, :] != 0, h2[2 * FD:3 * FD, :], zero))
    filter_emb = fsum / nf

    fa = jnp.concatenate([f.astype(bf16), ones1b], axis=0)
    hs = _dot(m1_s[...], fa)
    zero32 = jnp.zeros_like(hs[0:ES, :])
    hist_sum = (jnp.where(m[0:1, :] != 0, hs[0:ES, :], zero32)
                + jnp.where(m[1:2, :] != 0, hs[ES:2 * ES, :], zero32)
                + jnp.where(m[2:3, :] != 0, hs[2 * ES:3 * ES, :], zero32))
    hist_emb = hist_sum / nf
    samp = hs[96:128, :]

    i20 = _iota2((20, t), 0)
    i40 = _iota2((40, t), 0)
    i10 = _iota2((10, t), 0)
    acts3 = jnp.concatenate([(i20 == type_id).astype(bf16),
                             (i40 == join_id).astype(bf16),
                             (i10 == tab_id).astype(bf16),
                             filter_emb.astype(bf16),
                             samp.astype(bf16), hist_emb.astype(bf16),
                             ones1b], axis=0)
    o_ref[...] = _leaky(_dot(wfin_s[...], acts3))


@functools.partial(jax.jit, static_argnames=())
def kernel(feature, typeEmbed, tableEmbed, columnEmbed, opEmbed, joinEmbed,
           Wf, bf, Wf2, bf2, Ws, bs, Wh, bh, Wp, bp):
    b = feature.shape[0]
    f32 = jnp.float32

    ft = feature.T
    full = lambda s: pl.BlockSpec(s, lambda i: (0,) * len(s))
    scratch = lambda *s: pltpu.VMEM(s, jnp.bfloat16)
    out_t = pl.pallas_call(
        _body,
        grid=(b // TILE,),
        in_specs=[
            pl.BlockSpec((FEAT_DIM, TILE), lambda i: (0, i)),
            full((20, ES)), full((10, ES)), full((30, ES)), full((4, 4)),
            full((40, ES)), full((FD, FD)), full((FD,)), full((FD, FD)),
            full((FD,)), full((ES, 1000)), full((ES,)), full((ES, BIN)),
            full((ES,)), full((PD, PD)), full((PD,)),
        ],
        out_specs=pl.BlockSpec((PD, TILE), lambda i: (0, i)),
        out_shape=jax.ShapeDtypeStruct((PD, b), f32),
        scratch_shapes=[
            scratch(3 * FD, 106), scratch(3 * FD, 3 * FD + 1),
            scratch(128, FEAT_DIM + 1), scratch(PD, 172),
        ],
        compiler_params=pltpu.CompilerParams(
            dimension_semantics=("arbitrary",),
        ),
    )(ft, typeEmbed, tableEmbed, columnEmbed, opEmbed, joinEmbed,
      Wf, bf, Wf2, bf2, Ws, bs, Wh, bh, Wp, bp)
    return out_t.T
